# baseline (speedup 1.0000x reference)
import functools

import jax
import jax.numpy as jnp
from jax.experimental import pallas as pl
from jax.experimental.pallas import tpu as pltpu

_DEFAULT = jax.lax.Precision.DEFAULT


def _tri_ij(t, nj):
    i = jnp.int32(0)
    start = jnp.int32(0)
    for ii in range(1, nj):
        s_ii = ii * nj - (ii * (ii - 1)) // 2
        sel = t >= s_ii
        i = jnp.where(sel, ii, i)
        start = jnp.where(sel, s_ii - ii, start)
    return i, t - start


def _body(p_ref, ef_ref, ta_ref, tb_ref, adja_ref, adjb_ref, hv_ref, w_ref,
          bias_ref, out_ref, acc_ref, x_ref, *, nk, nj, bi, bj):
    t = pl.program_id(0)
    k = pl.program_id(1)
    i, j = _tri_ij(t, nj)

    @pl.when((t == 0) & (k == 0))
    def _():
        out_ref[...] = jnp.broadcast_to(bias_ref[...], out_ref.shape)

    @pl.when((i == 0) & (k == 0))
    def _():
        x_ref[pl.ds(j * bj, bj), :] = jax.lax.dot_general(
            hv_ref[...].astype(jnp.bfloat16),
            w_ref[...].astype(jnp.bfloat16),
            (((1,), (0,)), ((), ())),
            precision=_DEFAULT,
            preferred_element_type=jnp.float32).astype(jnp.bfloat16)

    vblock = (ef_ref[0:1, :] * p_ref[0, 0]
              + ef_ref[1:2, :] * p_ref[0, 1]
              + ef_ref[2:3, :] * p_ref[0, 2])
    a = (ta_ref[...] * vblock).astype(jnp.bfloat16)
    contrib = jax.lax.dot_general(
        a, tb_ref[...], (((1,), (1,)), ((), ())),
        precision=_DEFAULT, preferred_element_type=jnp.float32)

    @pl.when(k == 0)
    def _():
        acc_ref[...] = contrib

    @pl.when(k > 0)
    def _():
        acc_ref[...] += contrib

    @pl.when(k == nk - 1)
    def _():
        mult = acc_ref[...]
        adj = adja_ref[...]
        x_j = x_ref[pl.ds(j * bj, bj), :]
        ondiag = (i == j) & (jax.lax.broadcasted_iota(jnp.int32, (bi, bj), 0)
                             == jax.lax.broadcasted_iota(jnp.int32, (bi, bj), 1))
        c_row = jnp.where(ondiag, adj, adj * mult).astype(jnp.bfloat16)
        out_ref[pl.ds(i * bi, bi), :] += jax.lax.dot_general(
            c_row, x_j, (((1,), (0,)), ((), ())),
            precision=_DEFAULT, preferred_element_type=jnp.float32)

        @pl.when(i < j)
        def _():
            mult_t = mult.astype(jnp.bfloat16).T
            c_col = (adjb_ref[...] * mult_t.astype(jnp.float32)
                     ).astype(jnp.bfloat16)
            x_i = x_ref[pl.ds(i * bi, bi), :]
            out_ref[pl.ds(j * bj, bj), :] += jax.lax.dot_general(
                c_col, x_i, (((1,), (0,)), ((), ())),
                precision=_DEFAULT, preferred_element_type=jnp.float32)


def kernel(H_v, edge_features, adj_e, T, weight, bias, p):
    n, d = H_v.shape
    e = T.shape[1]
    bi = min(1024, n)
    bj = bi
    bk = min(1024, e)
    nj = n // bj
    nk = e // bk
    nt = (nj * (nj + 1)) // 2
    grid = (nt, nk)

    ef_t = edge_features.T
    bias2 = bias.reshape(1, d)

    def im_ta(t, k):
        i, _ = _tri_ij(t, nj)
        return (i, k)

    def im_tb(t, k):
        _, j = _tri_ij(t, nj)
        return (j, k)

    def im_adja(t, k):
        i, j = _tri_ij(t, nj)
        return (i, j)

    def im_adjb(t, k):
        i, j = _tri_ij(t, nj)
        return (j, i)

    def im_hv(t, k):
        _, j = _tri_ij(t, nj)
        return (j, 0)

    return pl.pallas_call(
        functools.partial(_body, nk=nk, nj=nj, bi=bi, bj=bj),
        grid=grid,
        in_specs=[
            pl.BlockSpec((1, 3), lambda t, k: (0, 0)),
            pl.BlockSpec((3, bk), lambda t, k: (0, k)),
            pl.BlockSpec((bi, bk), im_ta),
            pl.BlockSpec((bj, bk), im_tb),
            pl.BlockSpec((bi, bj), im_adja),
            pl.BlockSpec((bj, bi), im_adjb),
            pl.BlockSpec((bj, d), im_hv),
            pl.BlockSpec((d, d), lambda t, k: (0, 0)),
            pl.BlockSpec((1, d), lambda t, k: (0, 0)),
        ],
        out_specs=pl.BlockSpec((n, d), lambda t, k: (0, 0)),
        out_shape=jax.ShapeDtypeStruct((n, d), jnp.float32),
        scratch_shapes=[
            pltpu.VMEM((bi, bj), jnp.float32),
            pltpu.VMEM((n, d), jnp.bfloat16),
        ],
        compiler_params=pltpu.CompilerParams(
            dimension_semantics=("arbitrary", "arbitrary")),
    )(p, ef_t, T, T, adj_e, adj_e, H_v, weight, bias2)


# speedup vs baseline: 1.0118x; 1.0118x over previous
import functools

import jax
import jax.numpy as jnp
from jax.experimental import pallas as pl
from jax.experimental.pallas import tpu as pltpu

_DEFAULT = jax.lax.Precision.DEFAULT


def _tri_ij(t, nj):
    i = jnp.int32(0)
    start = jnp.int32(0)
    for ii in range(1, nj):
        s_ii = ii * nj - (ii * (ii - 1)) // 2
        sel = t >= s_ii
        i = jnp.where(sel, ii, i)
        start = jnp.where(sel, s_ii - ii, start)
    return i, t - start


def _body(p_ref, ef_ref, ta_ref, tb_ref, adja_ref, adjb_ref, hv_ref, w_ref,
          bias_ref, out_ref, acc_ref, x_ref, *, nk, nj, bi, bj):
    t = pl.program_id(0)
    k = pl.program_id(1)
    i, j = _tri_ij(t, nj)

    @pl.when((t == 0) & (k == 0))
    def _():
        out_ref[...] = jnp.broadcast_to(bias_ref[...], out_ref.shape)

    @pl.when((i == 0) & (k == 0))
    def _():
        x_ref[pl.ds(j * bj, bj), :] = jax.lax.dot_general(
            hv_ref[...].astype(jnp.bfloat16),
            w_ref[...].astype(jnp.bfloat16),
            (((1,), (0,)), ((), ())),
            precision=_DEFAULT,
            preferred_element_type=jnp.float32).astype(jnp.bfloat16)

    vblock = (ef_ref[0:1, :] * p_ref[0, 0]
              + ef_ref[1:2, :] * p_ref[0, 1]
              + ef_ref[2:3, :] * p_ref[0, 2])
    a = (ta_ref[...].astype(jnp.float32) * vblock).astype(jnp.bfloat16)
    contrib = jax.lax.dot_general(
        a, tb_ref[...], (((1,), (1,)), ((), ())),
        precision=_DEFAULT, preferred_element_type=jnp.float32)

    @pl.when(k == 0)
    def _():
        acc_ref[...] = contrib

    @pl.when((k > 0) & (k < nk - 1))
    def _():
        acc_ref[...] += contrib

    @pl.when(k == nk - 1)
    def _():
        mult = acc_ref[...] + contrib
        adj = adja_ref[...]
        x_j = x_ref[pl.ds(j * bj, bj), :]
        ondiag = (i == j) & (jax.lax.broadcasted_iota(jnp.int32, (bi, bj), 0)
                             == jax.lax.broadcasted_iota(jnp.int32, (bi, bj), 1))
        c_row = jnp.where(ondiag, adj, adj * mult).astype(jnp.bfloat16)
        out_ref[pl.ds(i * bi, bi), :] += jax.lax.dot_general(
            c_row, x_j, (((1,), (0,)), ((), ())),
            precision=_DEFAULT, preferred_element_type=jnp.float32)

        @pl.when(i < j)
        def _():
            mult_t = mult.astype(jnp.bfloat16).T
            c_col = (adjb_ref[...] * mult_t.astype(jnp.float32)
                     ).astype(jnp.bfloat16)
            x_i = x_ref[pl.ds(i * bi, bi), :]
            out_ref[pl.ds(j * bj, bj), :] += jax.lax.dot_general(
                c_col, x_i, (((1,), (0,)), ((), ())),
                precision=_DEFAULT, preferred_element_type=jnp.float32)


def kernel(H_v, edge_features, adj_e, T, weight, bias, p):
    n, d = H_v.shape
    e = T.shape[1]
    bi = min(1024, n)
    bj = bi
    bk = min(2048, e)
    nj = n // bj
    nk = e // bk
    nt = (nj * (nj + 1)) // 2
    grid = (nt, nk)

    T_bf = T.astype(jnp.bfloat16)
    ef_t = edge_features.T
    bias2 = bias.reshape(1, d)

    def im_ta(t, k):
        i, _ = _tri_ij(t, nj)
        return (i, k)

    def im_tb(t, k):
        _, j = _tri_ij(t, nj)
        return (j, k)

    def im_adja(t, k):
        i, j = _tri_ij(t, nj)
        return (i, j)

    def im_adjb(t, k):
        i, j = _tri_ij(t, nj)
        return (j, i)

    def im_hv(t, k):
        _, j = _tri_ij(t, nj)
        return (j, 0)

    return pl.pallas_call(
        functools.partial(_body, nk=nk, nj=nj, bi=bi, bj=bj),
        grid=grid,
        in_specs=[
            pl.BlockSpec((1, 3), lambda t, k: (0, 0)),
            pl.BlockSpec((3, bk), lambda t, k: (0, k)),
            pl.BlockSpec((bi, bk), im_ta),
            pl.BlockSpec((bj, bk), im_tb),
            pl.BlockSpec((bi, bj), im_adja),
            pl.BlockSpec((bj, bi), im_adjb),
            pl.BlockSpec((bj, d), im_hv),
            pl.BlockSpec((d, d), lambda t, k: (0, 0)),
            pl.BlockSpec((1, d), lambda t, k: (0, 0)),
        ],
        out_specs=pl.BlockSpec((n, d), lambda t, k: (0, 0)),
        out_shape=jax.ShapeDtypeStruct((n, d), jnp.float32),
        scratch_shapes=[
            pltpu.VMEM((bi, bj), jnp.float32),
            pltpu.VMEM((n, d), jnp.bfloat16),
        ],
        compiler_params=pltpu.CompilerParams(
            dimension_semantics=("arbitrary", "arbitrary")),
    )(p, ef_t, T_bf, T_bf, adj_e, adj_e, H_v, weight, bias2)
